# baseline (speedup 1.0000x reference)
import functools

import jax
import jax.numpy as jnp
from jax import lax
from jax.experimental import pallas as pl
from jax.experimental.pallas import tpu as pltpu
from jax.experimental.pallas import tpu_sc as plsc

_H = 384
_W = 384
_C = 4
_KS = 5
_OC = 192
_DYSTRIDE = 32
_KDIM = _KS * _DYSTRIDE
_RB = 8
_NB = _H // _RB
_N = _H * _W
_KSEL = _N // 2

_NWORK = 16
_PER = _N // _NWORK
_NV = _PER // 16


def _conv_body(xpad_ref, w1_ref, b1_ref, w2_ref, b2_ref, out_ref, patch_ref):
    patch_ref[...] = jnp.zeros((_KDIM, _W), jnp.bfloat16)
    for dyp in range(4):
        for c in range(_C):
            row = xpad_ref[dyp, c, :].reshape(1, _W + 4)
            for dx in range(_KS):
                base = (dyp + 1) * _DYSTRIDE + c * _KS + dx
                patch_ref[base:base + 1, :] = row[:, dx:dx + _W]

    w1 = w1_ref[...]
    b1 = b1_ref[...]
    w2 = w2_ref[...].astype(jnp.float32)
    b2v = jnp.sum(b2_ref[...])

    def block_step(blk, _):
        for j in range(_RB):
            patch_ref[0:_KDIM - _DYSTRIDE, :] = patch_ref[_DYSTRIDE:_KDIM, :]
            for c in range(_C):
                row = xpad_ref[pl.ds(blk * _RB + j + 4, 1), c, :]
                for dx in range(_KS):
                    base = 4 * _DYSTRIDE + c * _KS + dx
                    patch_ref[base:base + 1, :] = row[:, dx:dx + _W]
            f1 = lax.dot_general(w1, patch_ref[...], (((1,), (0,)), ((), ())),
                                 preferred_element_type=jnp.float32)
            f1 = f1 + b1
            f1b = f1.astype(jnp.bfloat16).astype(jnp.float32)
            f2 = jnp.sum(f1b * w2, axis=0, keepdims=True) + b2v
            ui = lax.bitcast_convert_type(f2, jnp.int32)
            out_ref[pl.ds(blk, 1), j, :] = jnp.where(
                ui < 0, ui ^ jnp.int32(0x7FFFFFFF), ui)
        return 0

    lax.fori_loop(0, _NB, block_step, 0)


@functools.partial(
    pl.kernel,
    out_type=jax.ShapeDtypeStruct((_N,), jnp.float32),
    mesh=plsc.VectorSubcoreMesh(core_axis_name="c", subcore_axis_name="s",
                                num_cores=1),
    scratch_types=[
        pltpu.VMEM((_PER,), jnp.float32),
        pltpu.VMEM((_PER,), jnp.int32),
        pltpu.VMEM((16,), jnp.int32),
        pltpu.VMEM((32,), jnp.int32),
        pltpu.VMEM((16 * _NWORK,), jnp.int32),
        pltpu.VMEM_SHARED((16 * _NWORK,), jnp.int32),
    ],
)
def _sc_select(keys_hbm, out_hbm, fvm, kvm, avm, cvm, rdvm, shared):
    wid = lax.axis_index("s")
    base = wid * _PER
    pltpu.sync_copy(keys_hbm.at[pl.ds(base, _PER)], kvm)

    signbit = jnp.int32(-2147483648)
    one = jnp.int32(1)
    zero = jnp.int32(0)
    ksel = jnp.full((16,), _KSEL, jnp.int32)

    pu = jnp.zeros((16,), jnp.int32)
    for bitpos in range(31, -1, -1):
        bitc = jnp.int32(-2147483648 if bitpos == 31 else 1 << bitpos)
        candu = pu | bitc
        cand = candu ^ signbit

        avm[...] = jnp.zeros((16,), jnp.int32)

        def cnt_step(v, _):
            kv = kvm[pl.ds(v * 16, 16)]
            avm[...] = avm[...] + jnp.where(kv >= cand, one, zero)
            return 0
        lax.fori_loop(0, _NV, cnt_step, 0, unroll=8)

        pltpu.sync_copy(avm, shared.at[pl.ds(wid * 16, 16)])
        plsc.subcore_barrier()
        pltpu.sync_copy(shared, rdvm)
        plsc.subcore_barrier()
        acc16 = rdvm[pl.ds(0, 16)]
        for w in range(1, _NWORK):
            acc16 = acc16 + rdvm[pl.ds(w * 16, 16)]
        cvm[pl.ds(0, 16)] = acc16
        cvm[pl.ds(16, 16)] = acc16
        total = cvm[pl.ds(0, 16)]
        for o in range(1, 16):
            total = total + cvm[pl.ds(o, 16)]
        pu = jnp.where(total >= ksel, candu, pu)

    thr = pu ^ signbit

    def mask_step(i, _):
        kv = kvm[pl.ds(i * 16, 16)]
        fvm[pl.ds(i * 16, 16)] = jnp.where(
            kv >= thr, jnp.float32(1.0), jnp.float32(0.0))
        return 0
    lax.fori_loop(0, _NV, mask_step, 0, unroll=8)
    pltpu.sync_copy(fvm, out_hbm.at[pl.ds(base, _PER)])


def kernel(ego_psm, cav_psm, W1, b1, W2, b2):
    x = jnp.concatenate([ego_psm, cav_psm], axis=0)
    xpad = jnp.pad(x, ((0, 0), (2, 2), (2, 2))).astype(jnp.bfloat16)
    xpad_t = jnp.transpose(xpad, (1, 0, 2))
    w1p = jnp.concatenate(
        [jnp.pad(W1[:, :, dy, :].reshape(_OC, _C * _KS), ((0, 0), (0, _DYSTRIDE - _C * _KS)))
         for dy in range(_KS)], axis=1).astype(jnp.bfloat16)
    b1r = b1.reshape(_OC, 1)
    w2r = W2.reshape(_OC, 1).astype(jnp.bfloat16)
    b2r = b2.reshape(1, 1)
    keys = pl.pallas_call(
        _conv_body,
        out_shape=jax.ShapeDtypeStruct((_NB, _RB, _W), jnp.int32),
        scratch_shapes=[pltpu.VMEM((_KDIM, _W), jnp.bfloat16)],
    )(xpad_t, w1p, b1r, w2r, b2r)
    mask = _sc_select(keys.reshape(_N))
    return mask.reshape(1, _H, _W)


# speedup vs baseline: 1.9138x; 1.9138x over previous
import functools

import jax
import jax.numpy as jnp
from jax import lax
from jax.experimental import pallas as pl
from jax.experimental.pallas import tpu as pltpu
from jax.experimental.pallas import tpu_sc as plsc

_H = 384
_W = 384
_C = 4
_KS = 5
_OC = 192
_DYSTRIDE = 32
_KDIM = _KS * _DYSTRIDE
_RB = 8
_NB = _H // _RB
_N = _H * _W
_KSEL = _N // 2

_NWORK = 16
_PER = _N // _NWORK
_NV = _PER // 16


def _conv_body(xpad_ref, w1_ref, b1_ref, w2_ref, b2_ref, out_ref, pref_ref,
               patch_ref):
    patch_ref[...] = jnp.zeros((_KDIM, _W), jnp.bfloat16)
    for dyp in range(4):
        for c in range(_C):
            row = xpad_ref[dyp, c, :].reshape(1, _W + 4)
            for dx in range(_KS):
                base = (dyp + 1) * _DYSTRIDE + c * _KS + dx
                patch_ref[base:base + 1, :] = row[:, dx:dx + _W]

    w1 = w1_ref[...]
    b1 = b1_ref[...]
    w2 = w2_ref[...].astype(jnp.float32)
    b2v = jnp.sum(b2_ref[...])

    def block_step(blk, _):
        for j in range(_RB):
            patch_ref[0:_KDIM - _DYSTRIDE, :] = patch_ref[_DYSTRIDE:_KDIM, :]
            for c in range(_C):
                row = xpad_ref[pl.ds(blk * _RB + j + 4, 1), c, :]
                for dx in range(_KS):
                    base = 4 * _DYSTRIDE + c * _KS + dx
                    patch_ref[base:base + 1, :] = row[:, dx:dx + _W]
            f1 = lax.dot_general(w1, patch_ref[...], (((1,), (0,)), ((), ())),
                                 preferred_element_type=jnp.float32)
            f1 = f1 + b1
            f1b = f1.astype(jnp.bfloat16).astype(jnp.float32)
            f2 = jnp.sum(f1b * w2, axis=0, keepdims=True) + b2v
            ui = lax.bitcast_convert_type(f2, jnp.int32)
            out_ref[pl.ds(blk, 1), j, :] = jnp.where(
                ui < 0, ui ^ jnp.int32(0x7FFFFFFF), ui)
        return 0

    lax.fori_loop(0, _NB, block_step, 0)

    keys = out_ref[...]
    signbit = jnp.int32(-2147483648)
    pu = jnp.int32(0)
    for bitpos in range(31, 15, -1):
        bitc = jnp.int32(-2147483648 if bitpos == 31 else 1 << bitpos)
        candu = pu | bitc
        cand = candu ^ signbit
        cnt = jnp.sum((keys >= cand).astype(jnp.int32))
        pu = jnp.where(cnt >= _KSEL, candu, pu)
    pref_ref[0:1, :] = jnp.zeros((1, 16), jnp.int32) + pu


@functools.partial(
    pl.kernel,
    out_type=jax.ShapeDtypeStruct((_N,), jnp.float32),
    mesh=plsc.VectorSubcoreMesh(core_axis_name="c", subcore_axis_name="s",
                                num_cores=1),
    scratch_types=[
        pltpu.VMEM((_PER,), jnp.float32),
        pltpu.VMEM((_PER,), jnp.int32),
        pltpu.VMEM((16,), jnp.int32),
        pltpu.VMEM((32,), jnp.int32),
        pltpu.VMEM((16 * _NWORK,), jnp.int32),
        pltpu.VMEM_SHARED((2 * 16 * _NWORK,), jnp.int32),
    ],
)
def _sc_select(keys_hbm, pref_hbm, out_hbm, fvm, kvm, avm, cvm, rdvm, shared):
    wid = lax.axis_index("s")
    base = wid * _PER
    pltpu.sync_copy(keys_hbm.at[pl.ds(base, _PER)], kvm)
    pltpu.sync_copy(pref_hbm, avm)

    signbit = jnp.int32(-2147483648)
    one = jnp.int32(1)
    zero = jnp.int32(0)
    ksel = jnp.full((16,), _KSEL, jnp.int32)

    pu = avm[...]
    for ri, bitpos in enumerate(range(15, -1, -1)):
        bitc = jnp.int32(-2147483648 if bitpos == 31 else 1 << bitpos)
        candu = pu | bitc
        cand = candu ^ signbit

        def cnt_step(v, acc):
            kv = kvm[pl.ds(v * 16, 16)]
            return acc + jnp.where(kv >= cand, one, zero)
        avm[...] = lax.fori_loop(0, _NV, cnt_step,
                                 jnp.zeros((16,), jnp.int32), unroll=8)

        off = (ri % 2) * 16 * _NWORK
        pltpu.sync_copy(avm, shared.at[pl.ds(off + wid * 16, 16)])
        plsc.subcore_barrier()
        pltpu.sync_copy(shared.at[pl.ds(off, 16 * _NWORK)], rdvm)
        acc16 = rdvm[pl.ds(0, 16)]
        for w in range(1, _NWORK):
            acc16 = acc16 + rdvm[pl.ds(w * 16, 16)]
        cvm[pl.ds(0, 16)] = acc16
        cvm[pl.ds(16, 16)] = acc16
        total = cvm[pl.ds(0, 16)]
        for o in range(1, 16):
            total = total + cvm[pl.ds(o, 16)]
        pu = jnp.where(total >= ksel, candu, pu)

    thr = pu ^ signbit


    def mask_step(i, _):
        kv = kvm[pl.ds(i * 16, 16)]
        fvm[pl.ds(i * 16, 16)] = jnp.where(
            kv >= thr, jnp.float32(1.0), jnp.float32(0.0))
        return 0
    lax.fori_loop(0, _NV, mask_step, 0, unroll=8)
    pltpu.sync_copy(fvm, out_hbm.at[pl.ds(base, _PER)])


def kernel(ego_psm, cav_psm, W1, b1, W2, b2):
    x = jnp.concatenate([ego_psm, cav_psm], axis=0)
    xpad = jnp.pad(x, ((0, 0), (2, 2), (2, 2))).astype(jnp.bfloat16)
    xpad_t = jnp.transpose(xpad, (1, 0, 2))
    w1p = jnp.concatenate(
        [jnp.pad(W1[:, :, dy, :].reshape(_OC, _C * _KS), ((0, 0), (0, _DYSTRIDE - _C * _KS)))
         for dy in range(_KS)], axis=1).astype(jnp.bfloat16)
    b1r = b1.reshape(_OC, 1)
    w2r = W2.reshape(_OC, 1).astype(jnp.bfloat16)
    b2r = b2.reshape(1, 1)
    keys, pref = pl.pallas_call(
        _conv_body,
        out_shape=[jax.ShapeDtypeStruct((_NB, _RB, _W), jnp.int32),
                   jax.ShapeDtypeStruct((1, 16), jnp.int32)],
        scratch_shapes=[pltpu.VMEM((_KDIM, _W), jnp.bfloat16)],
    )(xpad_t, w1p, b1r, w2r, b2r)
    mask = _sc_select(keys.reshape(_N), pref.reshape(16))
    return mask.reshape(1, _H, _W)


# speedup vs baseline: 1.9224x; 1.0045x over previous
import functools

import jax
import jax.numpy as jnp
from jax import lax
from jax.experimental import pallas as pl
from jax.experimental.pallas import tpu as pltpu
from jax.experimental.pallas import tpu_sc as plsc

_H = 384
_W = 384
_C = 4
_KS = 5
_OC = 192
_DYSTRIDE = 32
_KDIM = _KS * _DYSTRIDE
_RB = 8
_NB = _H // _RB
_N = _H * _W
_KSEL = _N // 2

_NWORK = 16
_PER = _N // _NWORK
_NV = _PER // 16


def _conv_body(xpad_ref, w1_ref, b1_ref, w2_ref, b2_ref, out_ref, pref_ref,
               patch_ref):
    patch_ref[...] = jnp.zeros((_KDIM, _W), jnp.bfloat16)
    for dyp in range(4):
        for c in range(_C):
            row = xpad_ref[dyp, c, :].reshape(1, _W + 4)
            for dx in range(_KS):
                base = (dyp + 1) * _DYSTRIDE + c * _KS + dx
                patch_ref[base:base + 1, :] = row[:, dx:dx + _W]

    w1 = w1_ref[...]
    b1 = b1_ref[...]
    w2 = w2_ref[...].astype(jnp.float32)
    b2v = jnp.sum(b2_ref[...])

    def block_step(blk, _):
        for j in range(_RB):
            patch_ref[0:_KDIM - _DYSTRIDE, :] = patch_ref[_DYSTRIDE:_KDIM, :]
            for c in range(_C):
                row = xpad_ref[pl.ds(blk * _RB + j + 4, 1), c, :]
                for dx in range(_KS):
                    base = 4 * _DYSTRIDE + c * _KS + dx
                    patch_ref[base:base + 1, :] = row[:, dx:dx + _W]
            f1 = lax.dot_general(w1, patch_ref[...], (((1,), (0,)), ((), ())),
                                 preferred_element_type=jnp.float32)
            f1 = f1 + b1
            f1b = f1.astype(jnp.bfloat16).astype(jnp.float32)
            f2 = jnp.sum(f1b * w2, axis=0, keepdims=True) + b2v
            ui = lax.bitcast_convert_type(f2, jnp.int32)
            out_ref[pl.ds(blk, 1), j, :] = jnp.where(
                ui < 0, ui ^ jnp.int32(0x7FFFFFFF), ui)
        return 0

    lax.fori_loop(0, _NB, block_step, 0)

    keys = out_ref[...]
    signbit = jnp.int32(-2147483648)
    pu = jnp.int32(0)
    for bitpos in range(31, 11, -1):
        bitc = jnp.int32(-2147483648 if bitpos == 31 else 1 << bitpos)
        candu = pu | bitc
        cand = candu ^ signbit
        cnt = jnp.sum((keys >= cand).astype(jnp.int32))
        pu = jnp.where(cnt >= _KSEL, candu, pu)
    pref_ref[0:1, :] = jnp.zeros((1, 16), jnp.int32) + pu


@functools.partial(
    pl.kernel,
    out_type=jax.ShapeDtypeStruct((_N,), jnp.float32),
    mesh=plsc.VectorSubcoreMesh(core_axis_name="c", subcore_axis_name="s",
                                num_cores=1),
    scratch_types=[
        pltpu.VMEM((_PER,), jnp.float32),
        pltpu.VMEM((_PER,), jnp.int32),
        pltpu.VMEM((16,), jnp.int32),
        pltpu.VMEM((32,), jnp.int32),
        pltpu.VMEM((16 * _NWORK,), jnp.int32),
        pltpu.VMEM_SHARED((2 * 16 * _NWORK,), jnp.int32),
    ],
)
def _sc_select(keys_hbm, pref_hbm, out_hbm, fvm, kvm, avm, cvm, rdvm, shared):
    wid = lax.axis_index("s")
    base = wid * _PER
    pltpu.sync_copy(keys_hbm.at[pl.ds(base, _PER)], kvm)
    pltpu.sync_copy(pref_hbm, avm)

    signbit = jnp.int32(-2147483648)
    one = jnp.int32(1)
    zero = jnp.int32(0)
    ksel = jnp.full((16,), _KSEL, jnp.int32)

    pu = avm[...]
    for ri, bitpos in enumerate(range(11, -1, -1)):
        bitc = jnp.int32(-2147483648 if bitpos == 31 else 1 << bitpos)
        candu = pu | bitc
        cand = candu ^ signbit

        def cnt_step(v, acc):
            kv = kvm[pl.ds(v * 16, 16)]
            return acc + jnp.where(kv >= cand, one, zero)
        avm[...] = lax.fori_loop(0, _NV, cnt_step,
                                 jnp.zeros((16,), jnp.int32), unroll=16)

        off = (ri % 2) * 16 * _NWORK
        pltpu.sync_copy(avm, shared.at[pl.ds(off + wid * 16, 16)])
        plsc.subcore_barrier()
        pltpu.sync_copy(shared.at[pl.ds(off, 16 * _NWORK)], rdvm)
        acc16 = rdvm[pl.ds(0, 16)]
        for w in range(1, _NWORK):
            acc16 = acc16 + rdvm[pl.ds(w * 16, 16)]
        cvm[pl.ds(0, 16)] = acc16
        cvm[pl.ds(16, 16)] = acc16
        total = cvm[pl.ds(0, 16)]
        for o in range(1, 16):
            total = total + cvm[pl.ds(o, 16)]
        pu = jnp.where(total >= ksel, candu, pu)

    thr = pu ^ signbit


    def mask_step(i, _):
        kv = kvm[pl.ds(i * 16, 16)]
        fvm[pl.ds(i * 16, 16)] = jnp.where(
            kv >= thr, jnp.float32(1.0), jnp.float32(0.0))
        return 0
    lax.fori_loop(0, _NV, mask_step, 0, unroll=16)
    pltpu.sync_copy(fvm, out_hbm.at[pl.ds(base, _PER)])


def kernel(ego_psm, cav_psm, W1, b1, W2, b2):
    x = jnp.concatenate([ego_psm, cav_psm], axis=0)
    xpad = jnp.pad(x, ((0, 0), (2, 2), (2, 2))).astype(jnp.bfloat16)
    xpad_t = jnp.transpose(xpad, (1, 0, 2))
    w1p = jnp.concatenate(
        [jnp.pad(W1[:, :, dy, :].reshape(_OC, _C * _KS), ((0, 0), (0, _DYSTRIDE - _C * _KS)))
         for dy in range(_KS)], axis=1).astype(jnp.bfloat16)
    b1r = b1.reshape(_OC, 1)
    w2r = W2.reshape(_OC, 1).astype(jnp.bfloat16)
    b2r = b2.reshape(1, 1)
    keys, pref = pl.pallas_call(
        _conv_body,
        out_shape=[jax.ShapeDtypeStruct((_NB, _RB, _W), jnp.int32),
                   jax.ShapeDtypeStruct((1, 16), jnp.int32)],
        scratch_shapes=[pltpu.VMEM((_KDIM, _W), jnp.bfloat16)],
    )(xpad_t, w1p, b1r, w2r, b2r)
    mask = _sc_select(keys.reshape(_N), pref.reshape(16))
    return mask.reshape(1, _H, _W)
